# baseline (speedup 1.0000x reference)
import functools

import jax
import jax.numpy as jnp
from jax.experimental import pallas as pl
from jax.experimental.pallas import tpu as pltpu

_NBUF = 4


def _copy(adj_ref, abuf_ref, sems_ref, c, n_blocks, bm):
    blk = jax.lax.rem(c, n_blocks)
    slot = jax.lax.rem(c, _NBUF)
    return pltpu.make_async_copy(
        adj_ref.at[pl.ds(blk * bm, bm), :],
        abuf_ref.at[slot],
        sems_ref.at[slot],
    )


def _gcn_body(adj_ref, x_ref, w0_ref, w1_ref, b0_ref, b1_ref, out_ref,
              abuf_ref, xbuf_ref, s_ref, h_ref, cs_ref, sems_ref, xsem_ref,
              *, n_rows, bm, n_blocks):
    p = n_blocks
    total = 2 * p
    inv_n = 1.0 / n_rows

    def start(c):
        @pl.when(c < total)
        def _():
            _copy(adj_ref, abuf_ref, sems_ref, c, p, bm).start()

    xcopy = pltpu.make_async_copy(x_ref, xbuf_ref, xsem_ref)
    start(jnp.int32(0))
    xcopy.start()
    for c in range(1, _NBUF):
        start(jnp.int32(c))

    xcopy.wait()
    s_ref[...] = jnp.dot(xbuf_ref[...], w0_ref[...],
                         preferred_element_type=jnp.float32)

    def mm_loop(j0, b_ref):
        cs_ref[...] = jnp.zeros_like(cs_ref)

        def body(j, _):
            c = j0 + j
            _copy(adj_ref, abuf_ref, sems_ref, c, p, bm).wait()
            slot = jax.lax.rem(c, _NBUF)
            h = jnp.dot(abuf_ref[slot], s_ref[...],
                        preferred_element_type=jnp.float32)
            h = jnp.maximum(h + b_ref[...], 0.0)
            h_ref[pl.ds(j * bm, bm), :] = h
            cs_ref[...] += jnp.sum(h, axis=0, keepdims=True)
            start(c + _NBUF)
            return _

        jax.lax.fori_loop(0, p, body, None)

    def pnorm():
        x = h_ref[...] - cs_ref[...] * inv_n
        rn = jnp.sqrt(1e-6 + jnp.sum(x * x, axis=1, keepdims=True))
        return x / rn

    mm_loop(jnp.int32(0), b0_ref)
    s_ref[...] = jnp.dot(pnorm(), w1_ref[...],
                         preferred_element_type=jnp.float32)
    mm_loop(jnp.int32(p), b1_ref)
    out_ref[...] = pnorm()


def _pick_block(n, target):
    best = 8
    for bm in range(8, min(n, target) + 1, 8):
        if n % bm == 0:
            best = bm
    return best


def kernel(in_feature, adj, W0, b0, W1, b1):
    n, d = in_feature.shape
    bm = _pick_block(n, 200)
    p = n // bm

    vmem = lambda: pl.BlockSpec(memory_space=pltpu.MemorySpace.VMEM)

    return pl.pallas_call(
        functools.partial(_gcn_body, n_rows=n, bm=bm, n_blocks=p),
        in_specs=[
            pl.BlockSpec(memory_space=pltpu.MemorySpace.HBM),
            pl.BlockSpec(memory_space=pltpu.MemorySpace.HBM),
            vmem(), vmem(), vmem(), vmem(),
        ],
        out_specs=vmem(),
        out_shape=jax.ShapeDtypeStruct((n, d), jnp.float32),
        scratch_shapes=[
            pltpu.VMEM((_NBUF, bm, n), jnp.float32),
            pltpu.VMEM((n, d), jnp.float32),
            pltpu.VMEM((n, d), jnp.float32),
            pltpu.VMEM((n, d), jnp.float32),
            pltpu.VMEM((1, d), jnp.float32),
            pltpu.SemaphoreType.DMA((_NBUF,)),
            pltpu.SemaphoreType.DMA,
        ],
    )(adj, in_feature, W0, W1, b0.reshape(1, d), b1.reshape(1, d))


# speedup vs baseline: 1.0040x; 1.0040x over previous
import functools

import jax
import jax.numpy as jnp
from jax.experimental import pallas as pl
from jax.experimental.pallas import tpu as pltpu

_NBUF = 4


def _copy(adj_ref, abuf_ref, sems_ref, c, n_blocks, bm):
    blk = jax.lax.rem(c, n_blocks)
    slot = jax.lax.rem(c, _NBUF)
    return pltpu.make_async_copy(
        adj_ref.at[pl.ds(blk * bm, bm), :],
        abuf_ref.at[slot],
        sems_ref.at[slot],
    )


def _gcn_body(adj_ref, x_ref, w0_ref, w1_ref, b0_ref, b1_ref, out_ref,
              abuf_ref, s_ref, h_ref, cs_ref, sems_ref,
              *, n_rows, bm, n_blocks):
    p = n_blocks
    total = 2 * p
    inv_n = 1.0 / n_rows

    def start(c):
        @pl.when(c < total)
        def _():
            _copy(adj_ref, abuf_ref, sems_ref, c, p, bm).start()

    for c in range(_NBUF):
        start(jnp.int32(c))

    s_ref[...] = jnp.dot(x_ref[...], w0_ref[...],
                         preferred_element_type=jnp.float32)

    def mm_loop(j0, b_ref):
        cs_ref[...] = jnp.zeros_like(cs_ref)

        def body(j, _):
            c = j0 + j
            _copy(adj_ref, abuf_ref, sems_ref, c, p, bm).wait()
            slot = jax.lax.rem(c, _NBUF)
            h = jnp.dot(abuf_ref[slot], s_ref[...],
                        preferred_element_type=jnp.float32)
            h = jnp.maximum(h + b_ref[...], 0.0)
            h_ref[pl.ds(j * bm, bm), :] = h
            cs_ref[...] += jnp.sum(h, axis=0, keepdims=True)
            start(c + _NBUF)
            return _

        jax.lax.fori_loop(0, p, body, None)

    def pnorm():
        x = h_ref[...] - cs_ref[...] * inv_n
        rn = jnp.sqrt(1e-6 + jnp.sum(x * x, axis=1, keepdims=True))
        return x / rn

    mm_loop(jnp.int32(0), b0_ref)
    s_ref[...] = jnp.dot(pnorm(), w1_ref[...],
                         preferred_element_type=jnp.float32)
    mm_loop(jnp.int32(p), b1_ref)
    out_ref[...] = pnorm()


def _pick_block(n, target):
    best = 8
    for bm in range(8, min(n, target) + 1, 8):
        if n % bm == 0:
            best = bm
    return best


def kernel(in_feature, adj, W0, b0, W1, b1):
    n, d = in_feature.shape
    bm = _pick_block(n, 200)
    p = n // bm

    vmem = lambda: pl.BlockSpec(memory_space=pltpu.MemorySpace.VMEM)

    return pl.pallas_call(
        functools.partial(_gcn_body, n_rows=n, bm=bm, n_blocks=p),
        in_specs=[
            pl.BlockSpec(memory_space=pltpu.MemorySpace.HBM),
            vmem(), vmem(), vmem(), vmem(), vmem(),
        ],
        out_specs=vmem(),
        out_shape=jax.ShapeDtypeStruct((n, d), jnp.float32),
        scratch_shapes=[
            pltpu.VMEM((_NBUF, bm, n), jnp.float32),
            pltpu.VMEM((n, d), jnp.float32),
            pltpu.VMEM((n, d), jnp.float32),
            pltpu.VMEM((1, d), jnp.float32),
            pltpu.SemaphoreType.DMA((_NBUF,)),
        ],
    )(adj, in_feature, W0, W1, b0.reshape(1, d), b1.reshape(1, d))
